# Initial kernel scaffold; baseline (speedup 1.0000x reference)
import jax
import jax.numpy as jnp
from jax.experimental import pallas as pl


def kernel(inp, pos, neg, emb_w, emb_f_w, emb_r_w):
    raise NotImplementedError("write your pallas kernel here")


# speedup vs baseline: 3.4527x; 3.4527x over previous
import functools

import jax
import jax.numpy as jnp
from jax import lax
from jax.experimental import pallas as pl
from jax.experimental.pallas import tpu as pltpu
from jax.experimental.pallas import tpu_sc as plsc

B = 4096
D2 = 128
NCAND = 28
NW = 32
ROWS_PER_W = B // NW
R = 8
NCHUNK = ROWS_PER_W // R
NA = 16
NB = 32

_COLS_A = tuple(6 * w + k for w in range(8) for k in (1, 4))
_COLS_B = tuple(6 * w + k for w in range(8) for k in (2, 3, 5, 6))


def _sc_body(idxf_hbm, idxa_hbm, idxb_hbm, cat_hbm,
             embw_hbm, embf_hbm, embr_hbm, out_hbm,
             idxf_v, idxa_v, idxb_v, cat_v,
             f_v, a_v, b_v, er_v, pred_v, sem):
    wid = lax.axis_index("s") * 2 + lax.axis_index("c")
    base = wid * ROWS_PER_W

    def chunk(c, carry):
        row0 = base + c * R
        pltpu.sync_copy(idxf_hbm.at[pl.ds(row0, R)], idxf_v)
        pltpu.sync_copy(idxa_hbm.at[pl.ds(row0 * NA, R * NA)], idxa_v)
        pltpu.sync_copy(idxb_hbm.at[pl.ds(row0 * NB, R * NB)], idxb_v)
        pltpu.sync_copy(cat_hbm.at[pl.ds(row0 * NCAND, R * NCAND)], cat_v)
        h0 = pltpu.async_copy(embf_hbm.at[idxf_v], f_v, sem)
        h1 = pltpu.async_copy(embw_hbm.at[idxa_v], a_v, sem)
        h2 = pltpu.async_copy(embw_hbm.at[idxb_v.at[pl.ds(0, 128)]],
                              b_v.at[pl.ds(0, 128)], sem)
        h3 = pltpu.async_copy(embw_hbm.at[idxb_v.at[pl.ds(128, 128)]],
                              b_v.at[pl.ds(128, 128)], sem)
        h4 = pltpu.async_copy(embr_hbm.at[cat_v.at[pl.ds(0, 112)]],
                              er_v.at[pl.ds(0, 112)], sem)
        h5 = pltpu.async_copy(embr_hbm.at[cat_v.at[pl.ds(112, 112)]],
                              er_v.at[pl.ds(112, 112)], sem)
        for h in (h0, h1, h2, h3, h4, h5):
            h.wait()

        for r in range(R):
            accA = lax.fori_loop(
                r * NA + 1, (r + 1) * NA,
                lambda j, acc: tuple(acc[q] + a_v[j, pl.ds(q * 16, 16)]
                                     for q in range(4)),
                tuple(a_v[r * NA, pl.ds(q * 16, 16)] for q in range(4)))
            accB = lax.fori_loop(
                r * NB + 1, (r + 1) * NB,
                lambda j, acc: tuple(acc[q] + b_v[j, pl.ds(q * 16, 16)]
                                     for q in range(4)),
                tuple(b_v[r * NB, pl.ds(q * 16, 16)] for q in range(4)))
            vv = []
            for q in range(4):
                vv.append((f_v[r, pl.ds(q * 16, 16)]
                           + accA[q] * 0.125) * (1.0 / 3.0))
            for q in range(4):
                vv.append((f_v[r, pl.ds(64 + q * 16, 16)]
                           + accB[q] * 0.0625) * (1.0 / 3.0))

            lane = lax.iota(jnp.int32, 16)
            last = lane == 15

            def nbody(n, carry2, r=r, vv=vv, lane=lane, last=last):
                i = r * NCAND + n
                acc = er_v[i, pl.ds(0, 16)] * vv[0]
                for q in range(1, 8):
                    acc = acc + er_v[i, pl.ds(q * 16, 16)] * vv[q]
                tot = plsc.cumsum(acc)
                plsc.store_scatter(pred_v, [jnp.full((16,), i, jnp.int32)],
                                   tot, mask=last)
                return carry2
            lax.fori_loop(0, NCAND, nbody, 0)

        pltpu.sync_copy(pred_v, out_hbm.at[pl.ds(row0 * NCAND, R * NCAND)])
        return carry

    lax.fori_loop(0, NCHUNK, chunk, 0)


def _sc_pred(idxf, idxa, idxb, cat, emb_w, emb_f_w, emb_r_w):
    mesh = plsc.VectorSubcoreMesh(core_axis_name="c", subcore_axis_name="s")
    fn = functools.partial(
        pl.kernel,
        out_type=jax.ShapeDtypeStruct((B * NCAND,), jnp.float32),
        scratch_types=[
            pltpu.VMEM((R,), jnp.int32),
            pltpu.VMEM((R * NA,), jnp.int32),
            pltpu.VMEM((R * NB,), jnp.int32),
            pltpu.VMEM((R * NCAND,), jnp.int32),
            pltpu.VMEM((R, D2), jnp.float32),
            pltpu.VMEM((R * NA, 64), jnp.float32),
            pltpu.VMEM((R * NB, 64), jnp.float32),
            pltpu.VMEM((R * NCAND, D2), jnp.float32),
            pltpu.VMEM((R * NCAND,), jnp.float32),
            pltpu.SemaphoreType.DMA,
        ],
        mesh=mesh,
        compiler_params=pltpu.CompilerParams(needs_layout_passes=False,
                                             use_tc_tiling_on_sc=False),
    )(_sc_body)
    return fn(idxf, idxa, idxb, cat, emb_w, emb_f_w, emb_r_w).reshape(B, NCAND)


def _loss_body(pred_ref, out_ref):
    x = pred_ref[...]
    col = lax.broadcasted_iota(jnp.int32, x.shape, 1)
    label = (col < 3).astype(jnp.float32)
    p = jnp.clip(jax.nn.sigmoid(x), 1e-7, 1.0 - 1e-7)
    ll = label * jnp.log(p) + (1.0 - label) * jnp.log(1.0 - p)
    out_ref[0, 0] = -jnp.sum(ll) * (1.0 / (B * NCAND))


def _loss(pred):
    out = pl.pallas_call(
        _loss_body,
        out_shape=jax.ShapeDtypeStruct((1, 1), jnp.float32),
        out_specs=pl.BlockSpec(memory_space=pltpu.SMEM),
    )(pred)
    return out[0, 0]


def kernel(inp, pos, neg, emb_w, emb_f_w, emb_r_w):
    inp = inp.astype(jnp.int32)
    idxf = inp[:, 0]
    idxa = inp[:, jnp.array(_COLS_A, jnp.int32)].reshape(-1)
    idxb = inp[:, jnp.array(_COLS_B, jnp.int32)].reshape(-1)
    cat = jnp.concatenate([pos.astype(jnp.int32),
                           neg.astype(jnp.int32)], axis=1).reshape(-1)
    pred = _sc_pred(idxf, idxa, idxb, cat, emb_w, emb_f_w, emb_r_w)
    return _loss(pred)


# speedup vs baseline: 3.8677x; 1.1202x over previous
import functools

import jax
import jax.numpy as jnp
from jax import lax
from jax.experimental import pallas as pl
from jax.experimental.pallas import tpu as pltpu
from jax.experimental.pallas import tpu_sc as plsc

B = 4096
SEQ = 50
D2 = 128
NPOS = 3
NNEG = 25
NCAND = NPOS + NNEG
NW = 32
ROWS_PER_W = B // NW
R = 8
NCHUNK = ROWS_PER_W // R
NA = 16
NB = 32


def _sc_body(inp_hbm, pos_hbm, neg_hbm,
             embw_hbm, embf_hbm, embr_hbm, out_hbm,
             inp_v, pos_v, neg_v,
             idxf_v, idxa_v, idxb_v,
             f_v, a_v, b_v, erp_v, ern_v, pred_v,
             sem_g, sem_s):
    wid = lax.axis_index("s") * 2 + lax.axis_index("c")
    base = wid * ROWS_PER_W
    lane = lax.iota(jnp.int32, 16)
    last = lane == 15

    def stage(c, buf):
        row0 = base + c * R
        pltpu.async_copy(inp_hbm.at[pl.ds(row0 * SEQ, R * SEQ)],
                         inp_v.at[buf], sem_s)
        pltpu.async_copy(pos_hbm.at[pl.ds(row0 * NPOS, R * NPOS)],
                         pos_v.at[buf], sem_s)
        pltpu.async_copy(neg_hbm.at[pl.ds(row0 * NNEG, R * NNEG)],
                         neg_v.at[buf], sem_s)

    def wait_stage(buf):
        pltpu.make_async_copy(inp_hbm.at[pl.ds(0, R * SEQ)],
                              inp_v.at[buf], sem_s).wait()
        pltpu.make_async_copy(pos_hbm.at[pl.ds(0, R * NPOS)],
                              pos_v.at[buf], sem_s).wait()
        pltpu.make_async_copy(neg_hbm.at[pl.ds(0, R * NNEG)],
                              neg_v.at[buf], sem_s).wait()

    def fire(buf):
        bvec = jnp.full((16,), buf, jnp.int32)
        fpos = (lane & 7) * SEQ
        idxf_v[buf, pl.ds(0, 16)] = plsc.load_gather(inp_v, [bvec, fpos])
        for r in range(R):
            apos = r * SEQ + 3 * lane + 1
            idxa_v[buf, pl.ds(r * NA, 16)] = plsc.load_gather(
                inp_v, [bvec, apos])
            for h in range(2):
                j = lane + h * 16
                k = j & 3
                bpos = r * SEQ + 6 * (j >> 2) + 2 + k + (k >> 1)
                idxb_v[buf, pl.ds(r * NB + h * 16, 16)] = plsc.load_gather(
                    inp_v, [bvec, bpos])
        pltpu.async_copy(embf_hbm.at[idxf_v.at[buf, pl.ds(0, 8)]],
                         f_v.at[buf], sem_g)
        pltpu.async_copy(embw_hbm.at[idxa_v.at[buf]], a_v.at[buf], sem_g)
        pltpu.async_copy(embw_hbm.at[idxb_v.at[buf, pl.ds(0, 128)]],
                         b_v.at[buf, pl.ds(0, 128)], sem_g)
        pltpu.async_copy(embw_hbm.at[idxb_v.at[buf, pl.ds(128, 128)]],
                         b_v.at[buf, pl.ds(128, 128)], sem_g)
        pltpu.async_copy(embr_hbm.at[pos_v.at[buf]], erp_v.at[buf], sem_g)
        pltpu.async_copy(embr_hbm.at[neg_v.at[buf, pl.ds(0, 104)]],
                         ern_v.at[buf, pl.ds(0, 104)], sem_g)
        pltpu.async_copy(embr_hbm.at[neg_v.at[buf, pl.ds(104, 96)]],
                         ern_v.at[buf, pl.ds(104, 96)], sem_g)

    def wait_gathers(buf):
        pltpu.make_async_copy(embf_hbm.at[idxf_v.at[buf, pl.ds(0, 8)]],
                              f_v.at[buf], sem_g).wait()
        pltpu.make_async_copy(embw_hbm.at[idxa_v.at[buf]],
                              a_v.at[buf], sem_g).wait()
        pltpu.make_async_copy(embw_hbm.at[idxb_v.at[buf, pl.ds(0, 128)]],
                              b_v.at[buf, pl.ds(0, 128)], sem_g).wait()
        pltpu.make_async_copy(embw_hbm.at[idxb_v.at[buf, pl.ds(128, 128)]],
                              b_v.at[buf, pl.ds(128, 128)], sem_g).wait()
        pltpu.make_async_copy(embr_hbm.at[pos_v.at[buf]],
                              erp_v.at[buf], sem_g).wait()
        pltpu.make_async_copy(embr_hbm.at[neg_v.at[buf, pl.ds(0, 104)]],
                              ern_v.at[buf, pl.ds(0, 104)], sem_g).wait()
        pltpu.make_async_copy(embr_hbm.at[neg_v.at[buf, pl.ds(104, 96)]],
                              ern_v.at[buf, pl.ds(104, 96)], sem_g).wait()

    def compute(c, buf):
        row0 = base + c * R
        bvec = jnp.full((16,), buf, jnp.int32)
        for r in range(R):
            accA = lax.fori_loop(
                r * NA + 1, (r + 1) * NA,
                lambda j, acc: tuple(acc[q] + a_v[buf, j, pl.ds(q * 16, 16)]
                                     for q in range(4)),
                tuple(a_v[buf, r * NA, pl.ds(q * 16, 16)]
                      for q in range(4)),
                unroll=5)
            accB = lax.fori_loop(
                r * NB + 1, (r + 1) * NB,
                lambda j, acc: tuple(acc[q] + b_v[buf, j, pl.ds(q * 16, 16)]
                                     for q in range(4)),
                tuple(b_v[buf, r * NB, pl.ds(q * 16, 16)]
                      for q in range(4)),
                unroll=5)
            vv = []
            for q in range(4):
                vv.append((f_v[buf, r, pl.ds(q * 16, 16)]
                           + accA[q] * 0.125) * (1.0 / 3.0))
            for q in range(4):
                vv.append((f_v[buf, r, pl.ds(64 + q * 16, 16)]
                           + accB[q] * 0.0625) * (1.0 / 3.0))

            for n in range(NPOS):
                i = r * NPOS + n
                acc = erp_v[buf, i, pl.ds(0, 16)] * vv[0]
                for q in range(1, 8):
                    acc = acc + erp_v[buf, i, pl.ds(q * 16, 16)] * vv[q]
                tot = plsc.cumsum(acc)
                plsc.store_scatter(
                    pred_v,
                    [bvec, jnp.full((16,), r * NCAND + n, jnp.int32)],
                    tot, mask=last)

            def nbody(n, carry2, r=r, vv=vv):
                i = r * NNEG + n
                acc = ern_v[buf, i, pl.ds(0, 16)] * vv[0]
                for q in range(1, 8):
                    acc = acc + ern_v[buf, i, pl.ds(q * 16, 16)] * vv[q]
                tot = plsc.cumsum(acc)
                plsc.store_scatter(
                    pred_v,
                    [bvec, jnp.full((16,), r * NCAND + NPOS + n, jnp.int32)],
                    tot, mask=last)
                return carry2
            lax.fori_loop(0, NNEG, nbody, 0, unroll=5)

        pltpu.sync_copy(pred_v.at[buf],
                        out_hbm.at[pl.ds(row0 * NCAND, R * NCAND)])

    stage(0, 0)
    wait_stage(0)
    fire(0)
    stage(1, 1)

    def step(c, carry):
        buf = lax.rem(c, 2)
        wait_gathers(buf)

        @pl.when(c + 1 < NCHUNK)
        def _():
            wait_stage(1 - buf)
            fire(1 - buf)

        @pl.when(c + 2 < NCHUNK)
        def _():
            stage(c + 2, buf)

        compute(c, buf)
        return carry

    lax.fori_loop(0, NCHUNK, step, 0)


def _sc_pred(inp, pos, neg, emb_w, emb_f_w, emb_r_w):
    mesh = plsc.VectorSubcoreMesh(core_axis_name="c", subcore_axis_name="s")
    fn = functools.partial(
        pl.kernel,
        out_type=jax.ShapeDtypeStruct((B * NCAND,), jnp.float32),
        scratch_types=[
            pltpu.VMEM((2, R * SEQ), jnp.int32),
            pltpu.VMEM((2, R * NPOS), jnp.int32),
            pltpu.VMEM((2, R * NNEG), jnp.int32),
            pltpu.VMEM((2, 16), jnp.int32),
            pltpu.VMEM((2, R * NA), jnp.int32),
            pltpu.VMEM((2, R * NB), jnp.int32),
            pltpu.VMEM((2, R, D2), jnp.float32),
            pltpu.VMEM((2, R * NA, 64), jnp.float32),
            pltpu.VMEM((2, R * NB, 64), jnp.float32),
            pltpu.VMEM((2, R * NPOS, D2), jnp.float32),
            pltpu.VMEM((2, R * NNEG, D2), jnp.float32),
            pltpu.VMEM((2, R * NCAND), jnp.float32),
            pltpu.SemaphoreType.DMA,
            pltpu.SemaphoreType.DMA,
        ],
        mesh=mesh,
        compiler_params=pltpu.CompilerParams(needs_layout_passes=False,
                                             use_tc_tiling_on_sc=False),
    )(_sc_body)
    return fn(inp, pos, neg, emb_w, emb_f_w, emb_r_w).reshape(B, NCAND)


def _loss_body(pred_ref, out_ref):
    x = pred_ref[...]
    col = lax.broadcasted_iota(jnp.int32, x.shape, 1)
    label = (col < NPOS).astype(jnp.float32)
    p = jnp.clip(jax.nn.sigmoid(x), 1e-7, 1.0 - 1e-7)
    ll = label * jnp.log(p) + (1.0 - label) * jnp.log(1.0 - p)
    out_ref[0, 0] = -jnp.sum(ll) * (1.0 / (B * NCAND))


def _loss(pred):
    out = pl.pallas_call(
        _loss_body,
        out_shape=jax.ShapeDtypeStruct((1, 1), jnp.float32),
        out_specs=pl.BlockSpec(memory_space=pltpu.SMEM),
    )(pred)
    return out[0, 0]


def kernel(inp, pos, neg, emb_w, emb_f_w, emb_r_w):
    inp = inp.astype(jnp.int32).reshape(-1)
    pos = pos.astype(jnp.int32).reshape(-1)
    neg = neg.astype(jnp.int32).reshape(-1)
    pred = _sc_pred(inp, pos, neg, emb_w, emb_f_w, emb_r_w)
    return _loss(pred)


# speedup vs baseline: 4.8174x; 1.2456x over previous
import functools

import jax
import jax.numpy as jnp
from jax import lax
from jax.experimental import pallas as pl
from jax.experimental.pallas import tpu as pltpu
from jax.experimental.pallas import tpu_sc as plsc

B = 4096
SEQ = 50
D2 = 128
NPOS = 3
NNEG = 25
NCAND = NPOS + NNEG
NW = 32
ROWS_PER_W = B // NW
R = 8
NCHUNK = ROWS_PER_W // R
NPAIR = NCHUNK // 2
NA = 16
NB = 32


def _sc_body(inp_hbm, pos_hbm, neg_hbm,
             embw_hbm, embf_hbm, embr_hbm, out_hbm,
             inp0, pos0, neg0, idxf0, idxa0, idxb0,
             f0, a0, b0, erp0, ern0, pred0,
             inp1, pos1, neg1, idxf1, idxa1, idxb1,
             f1, a1, b1, erp1, ern1, pred1,
             sem_g, sem_s):
    wid = lax.axis_index("s") * 2 + lax.axis_index("c")
    base = wid * ROWS_PER_W
    lane = lax.iota(jnp.int32, 16)
    last = lane == 15

    bufs0 = (inp0, pos0, neg0, idxf0, idxa0, idxb0,
             f0, a0, b0, erp0, ern0, pred0)
    bufs1 = (inp1, pos1, neg1, idxf1, idxa1, idxb1,
             f1, a1, b1, erp1, ern1, pred1)

    def stage(c, bufs):
        inp_v, pos_v, neg_v = bufs[0], bufs[1], bufs[2]
        row0 = base + c * R
        pltpu.async_copy(inp_hbm.at[pl.ds(row0 * SEQ, R * SEQ)],
                         inp_v, sem_s)
        pltpu.async_copy(pos_hbm.at[pl.ds(row0 * NPOS, R * NPOS)],
                         pos_v, sem_s)
        pltpu.async_copy(neg_hbm.at[pl.ds(row0 * NNEG, R * NNEG)],
                         neg_v, sem_s)

    def wait_stage(bufs):
        inp_v, pos_v, neg_v = bufs[0], bufs[1], bufs[2]
        pltpu.make_async_copy(inp_hbm.at[pl.ds(0, R * SEQ)],
                              inp_v, sem_s).wait()
        pltpu.make_async_copy(pos_hbm.at[pl.ds(0, R * NPOS)],
                              pos_v, sem_s).wait()
        pltpu.make_async_copy(neg_hbm.at[pl.ds(0, R * NNEG)],
                              neg_v, sem_s).wait()

    def fire(bufs):
        (inp_v, pos_v, neg_v, idxf_v, idxa_v, idxb_v,
         f_v, a_v, b_v, erp_v, ern_v, _) = bufs
        idxf_v[pl.ds(0, 16)] = plsc.load_gather(inp_v, [(lane & 7) * SEQ])
        for r in range(R):
            idxa_v[pl.ds(r * NA, 16)] = plsc.load_gather(
                inp_v, [r * SEQ + 3 * lane + 1])
            for h in range(2):
                j = lane + h * 16
                k = j & 3
                idxb_v[pl.ds(r * NB + h * 16, 16)] = plsc.load_gather(
                    inp_v, [r * SEQ + 6 * (j >> 2) + 2 + k + (k >> 1)])
        pltpu.async_copy(embf_hbm.at[idxf_v.at[pl.ds(0, 8)]], f_v, sem_g)
        pltpu.async_copy(embw_hbm.at[idxa_v], a_v, sem_g)
        pltpu.async_copy(embw_hbm.at[idxb_v.at[pl.ds(0, 128)]],
                         b_v.at[pl.ds(0, 128)], sem_g)
        pltpu.async_copy(embw_hbm.at[idxb_v.at[pl.ds(128, 128)]],
                         b_v.at[pl.ds(128, 128)], sem_g)
        pltpu.async_copy(embr_hbm.at[pos_v], erp_v, sem_g)
        pltpu.async_copy(embr_hbm.at[neg_v.at[pl.ds(0, 104)]],
                         ern_v.at[pl.ds(0, 104)], sem_g)
        pltpu.async_copy(embr_hbm.at[neg_v.at[pl.ds(104, 96)]],
                         ern_v.at[pl.ds(104, 96)], sem_g)

    def wait_g(bufs):
        (_, pos_v, neg_v, idxf_v, idxa_v, idxb_v,
         f_v, a_v, b_v, erp_v, ern_v, _) = bufs
        pltpu.make_async_copy(embf_hbm.at[idxf_v.at[pl.ds(0, 8)]],
                              f_v, sem_g).wait()
        pltpu.make_async_copy(embw_hbm.at[idxa_v], a_v, sem_g).wait()
        pltpu.make_async_copy(embw_hbm.at[idxb_v.at[pl.ds(0, 128)]],
                              b_v.at[pl.ds(0, 128)], sem_g).wait()
        pltpu.make_async_copy(embw_hbm.at[idxb_v.at[pl.ds(128, 128)]],
                              b_v.at[pl.ds(128, 128)], sem_g).wait()
        pltpu.make_async_copy(embr_hbm.at[pos_v], erp_v, sem_g).wait()
        pltpu.make_async_copy(embr_hbm.at[neg_v.at[pl.ds(0, 104)]],
                              ern_v.at[pl.ds(0, 104)], sem_g).wait()
        pltpu.make_async_copy(embr_hbm.at[neg_v.at[pl.ds(104, 96)]],
                              ern_v.at[pl.ds(104, 96)], sem_g).wait()

    def compute(c, bufs):
        (_, _, _, _, _, _, f_v, a_v, b_v, erp_v, ern_v, pred_v) = bufs
        row0 = base + c * R

        def rowbody(r, carry):
            accA = [a_v[r * NA, pl.ds(q * 16, 16)] for q in range(4)]
            for j in range(1, NA):
                for q in range(4):
                    accA[q] = accA[q] + a_v[r * NA + j, pl.ds(q * 16, 16)]
            accB = [b_v[r * NB, pl.ds(q * 16, 16)] for q in range(4)]
            for j in range(1, NB):
                for q in range(4):
                    accB[q] = accB[q] + b_v[r * NB + j, pl.ds(q * 16, 16)]
            vv = []
            for q in range(4):
                vv.append((f_v[r, pl.ds(q * 16, 16)]
                           + accA[q] * 0.125) * (1.0 / 3.0))
            for q in range(4):
                vv.append((f_v[r, pl.ds(64 + q * 16, 16)]
                           + accB[q] * 0.0625) * (1.0 / 3.0))
            for n in range(NCAND):
                if n < NPOS:
                    er, i = erp_v, r * NPOS + n
                else:
                    er, i = ern_v, r * NNEG + (n - NPOS)
                acc = er[i, pl.ds(0, 16)] * vv[0]
                for q in range(1, 8):
                    acc = acc + er[i, pl.ds(q * 16, 16)] * vv[q]
                tot = plsc.cumsum(acc)
                plsc.store_scatter(
                    pred_v, [jnp.full((16,), r * NCAND + n, jnp.int32)],
                    tot, mask=last)
            return carry

        lax.fori_loop(0, R, rowbody, 0)
        pltpu.sync_copy(pred_v, out_hbm.at[pl.ds(row0 * NCAND, R * NCAND)])

    stage(0, bufs0)
    wait_stage(bufs0)
    fire(bufs0)
    stage(1, bufs1)

    def pairstep(p, carry):
        c0 = 2 * p
        wait_stage(bufs1)
        fire(bufs1)

        @pl.when(p < NPAIR - 1)
        def _():
            stage(c0 + 2, bufs0)

        wait_g(bufs0)
        compute(c0, bufs0)

        @pl.when(p < NPAIR - 1)
        def _():
            wait_stage(bufs0)
            fire(bufs0)
            stage(c0 + 3, bufs1)

        wait_g(bufs1)
        compute(c0 + 1, bufs1)
        return carry

    lax.fori_loop(0, NPAIR, pairstep, 0)


def _sc_pred(inp, pos, neg, emb_w, emb_f_w, emb_r_w):
    mesh = plsc.VectorSubcoreMesh(core_axis_name="c", subcore_axis_name="s")
    buf_set = [
        pltpu.VMEM((R * SEQ,), jnp.int32),
        pltpu.VMEM((R * NPOS,), jnp.int32),
        pltpu.VMEM((R * NNEG,), jnp.int32),
        pltpu.VMEM((16,), jnp.int32),
        pltpu.VMEM((R * NA,), jnp.int32),
        pltpu.VMEM((R * NB,), jnp.int32),
        pltpu.VMEM((R, D2), jnp.float32),
        pltpu.VMEM((R * NA, 64), jnp.float32),
        pltpu.VMEM((R * NB, 64), jnp.float32),
        pltpu.VMEM((R * NPOS, D2), jnp.float32),
        pltpu.VMEM((R * NNEG, D2), jnp.float32),
        pltpu.VMEM((R * NCAND,), jnp.float32),
    ]
    fn = functools.partial(
        pl.kernel,
        out_type=jax.ShapeDtypeStruct((B * NCAND,), jnp.float32),
        scratch_types=buf_set + buf_set + [
            pltpu.SemaphoreType.DMA,
            pltpu.SemaphoreType.DMA,
        ],
        mesh=mesh,
        compiler_params=pltpu.CompilerParams(needs_layout_passes=False,
                                             use_tc_tiling_on_sc=False),
    )(_sc_body)
    return fn(inp, pos, neg, emb_w, emb_f_w, emb_r_w)


def _loss_body(pred_ref, out_ref):
    x = pred_ref[...]
    i = lax.broadcasted_iota(jnp.int32, x.shape, 0)
    j = lax.broadcasted_iota(jnp.int32, x.shape, 1)
    col = (i * 128 + j) % NCAND
    label = (col < NPOS).astype(jnp.float32)
    p = jnp.clip(jax.nn.sigmoid(x), 1e-7, 1.0 - 1e-7)
    ll = label * jnp.log(p) + (1.0 - label) * jnp.log(1.0 - p)
    out_ref[0, 0] = -jnp.sum(ll) * (1.0 / (B * NCAND))


def _loss(pred_flat):
    out = pl.pallas_call(
        _loss_body,
        out_shape=jax.ShapeDtypeStruct((1, 1), jnp.float32),
        out_specs=pl.BlockSpec(memory_space=pltpu.SMEM),
    )(pred_flat.reshape(B * NCAND // 128, 128))
    return out[0, 0]


def kernel(inp, pos, neg, emb_w, emb_f_w, emb_r_w):
    inp = inp.astype(jnp.int32).reshape(-1)
    pos = pos.astype(jnp.int32).reshape(-1)
    neg = neg.astype(jnp.int32).reshape(-1)
    pred = _sc_pred(inp, pos, neg, emb_w, emb_f_w, emb_r_w)
    return _loss(pred)


# speedup vs baseline: 6.1267x; 1.2718x over previous
import functools

import jax
import jax.numpy as jnp
from jax import lax
from jax.experimental import pallas as pl
from jax.experimental.pallas import tpu as pltpu
from jax.experimental.pallas import tpu_sc as plsc

B = 4096
SEQ = 50
D2 = 128
NPOS = 3
NNEG = 25
NCAND = NPOS + NNEG
NW = 32
ROWS_PER_W = B // NW
R = 8
NCHUNK = ROWS_PER_W // R
NPAIR = NCHUNK // 2
NA = 16
NB = 32


def _sc_body(inp_hbm, pos_hbm, neg_hbm,
             embw_hbm, embf_hbm, embr_hbm, out_hbm,
             inp0, pos0, neg0, idxf0, idxa0, idxb0,
             f0, a0, b0, erp0, ern0, pred0,
             inp1, pos1, neg1, idxf1, idxa1, idxb1,
             f1, a1, b1, erp1, ern1, pred1,
             sem_g, sem_s):
    wid = lax.axis_index("s") * 2 + lax.axis_index("c")
    base = wid * ROWS_PER_W
    lane = lax.iota(jnp.int32, 16)
    last = lane == 15

    bufs0 = (inp0, pos0, neg0, idxf0, idxa0, idxb0,
             f0, a0, b0, erp0, ern0, pred0)
    bufs1 = (inp1, pos1, neg1, idxf1, idxa1, idxb1,
             f1, a1, b1, erp1, ern1, pred1)

    def stage(c, bufs):
        inp_v, pos_v, neg_v = bufs[0], bufs[1], bufs[2]
        row0 = base + c * R
        pltpu.async_copy(inp_hbm.at[pl.ds(row0 * SEQ, R * SEQ)],
                         inp_v, sem_s)
        pltpu.async_copy(pos_hbm.at[pl.ds(row0 * NPOS, R * NPOS)],
                         pos_v, sem_s)
        pltpu.async_copy(neg_hbm.at[pl.ds(row0 * NNEG, R * NNEG)],
                         neg_v, sem_s)

    def wait_stage(bufs):
        inp_v, pos_v, neg_v = bufs[0], bufs[1], bufs[2]
        pltpu.make_async_copy(inp_hbm.at[pl.ds(0, R * SEQ)],
                              inp_v, sem_s).wait()
        pltpu.make_async_copy(pos_hbm.at[pl.ds(0, R * NPOS)],
                              pos_v, sem_s).wait()
        pltpu.make_async_copy(neg_hbm.at[pl.ds(0, R * NNEG)],
                              neg_v, sem_s).wait()

    def fire(bufs):
        (inp_v, pos_v, neg_v, idxf_v, idxa_v, idxb_v,
         f_v, a_v, b_v, erp_v, ern_v, _) = bufs
        idxf_v[pl.ds(0, 16)] = plsc.load_gather(inp_v, [(lane & 7) * SEQ])
        for r in range(R):
            idxa_v[pl.ds(r * NA, 16)] = plsc.load_gather(
                inp_v, [r * SEQ + 3 * lane + 1])
            for h in range(2):
                j = lane + h * 16
                k = j & 3
                idxb_v[pl.ds(r * NB + h * 16, 16)] = plsc.load_gather(
                    inp_v, [r * SEQ + 6 * (j >> 2) + 2 + k + (k >> 1)])
        pltpu.async_copy(embf_hbm.at[idxf_v.at[pl.ds(0, 8)]], f_v, sem_g)
        pltpu.async_copy(embw_hbm.at[idxa_v], a_v, sem_g)
        pltpu.async_copy(embw_hbm.at[idxb_v.at[pl.ds(0, 128)]],
                         b_v.at[pl.ds(0, 128)], sem_g)
        pltpu.async_copy(embw_hbm.at[idxb_v.at[pl.ds(128, 128)]],
                         b_v.at[pl.ds(128, 128)], sem_g)
        pltpu.async_copy(embr_hbm.at[pos_v], erp_v, sem_g)
        pltpu.async_copy(embr_hbm.at[neg_v.at[pl.ds(0, 104)]],
                         ern_v.at[pl.ds(0, 104)], sem_g)
        pltpu.async_copy(embr_hbm.at[neg_v.at[pl.ds(104, 96)]],
                         ern_v.at[pl.ds(104, 96)], sem_g)

    def wait_g(bufs):
        (_, pos_v, neg_v, idxf_v, idxa_v, idxb_v,
         f_v, a_v, b_v, erp_v, ern_v, _) = bufs
        pltpu.make_async_copy(embf_hbm.at[idxf_v.at[pl.ds(0, 8)]],
                              f_v, sem_g).wait()
        pltpu.make_async_copy(embw_hbm.at[idxa_v], a_v, sem_g).wait()
        pltpu.make_async_copy(embw_hbm.at[idxb_v.at[pl.ds(0, 128)]],
                              b_v.at[pl.ds(0, 128)], sem_g).wait()
        pltpu.make_async_copy(embw_hbm.at[idxb_v.at[pl.ds(128, 128)]],
                              b_v.at[pl.ds(128, 128)], sem_g).wait()
        pltpu.make_async_copy(embr_hbm.at[pos_v], erp_v, sem_g).wait()
        pltpu.make_async_copy(embr_hbm.at[neg_v.at[pl.ds(0, 104)]],
                              ern_v.at[pl.ds(0, 104)], sem_g).wait()
        pltpu.make_async_copy(embr_hbm.at[neg_v.at[pl.ds(104, 96)]],
                              ern_v.at[pl.ds(104, 96)], sem_g).wait()

    def compute(c, bufs):
        (_, _, _, _, _, _, f_v, a_v, b_v, erp_v, ern_v, pred_v) = bufs
        row0 = base + c * R

        def tree_sum(vs):
            while len(vs) > 1:
                nxt = [vs[i] + vs[i + 1] for i in range(0, len(vs) - 1, 2)]
                if len(vs) % 2:
                    nxt.append(vs[-1])
                vs = nxt
            return vs[0]

        def rowbody(r, carry):
            accA = [tree_sum([a_v[r * NA + j, pl.ds(q * 16, 16)]
                              for j in range(NA)]) for q in range(4)]
            accB = [tree_sum([b_v[r * NB + j, pl.ds(q * 16, 16)]
                              for j in range(NB)]) for q in range(4)]
            vv = []
            for q in range(4):
                vv.append((f_v[r, pl.ds(q * 16, 16)]
                           + accA[q] * 0.125) * (1.0 / 3.0))
            for q in range(4):
                vv.append((f_v[r, pl.ds(64 + q * 16, 16)]
                           + accB[q] * 0.0625) * (1.0 / 3.0))
            for g in range(0, NCAND, 4):
                accs = []
                for n in range(g, min(g + 4, NCAND)):
                    if n < NPOS:
                        er, i = erp_v, r * NPOS + n
                    else:
                        er, i = ern_v, r * NNEG + (n - NPOS)
                    accs.append(tree_sum(
                        [er[i, pl.ds(q * 16, 16)] * vv[q]
                         for q in range(8)]))
                tots = [plsc.cumsum(a) for a in accs]
                for k, tot in enumerate(tots):
                    plsc.store_scatter(
                        pred_v,
                        [jnp.full((16,), r * NCAND + g + k, jnp.int32)],
                        tot, mask=last)
            return carry

        lax.fori_loop(0, R, rowbody, 0)
        pltpu.sync_copy(pred_v, out_hbm.at[pl.ds(row0 * NCAND, R * NCAND)])

    stage(0, bufs0)
    wait_stage(bufs0)
    fire(bufs0)
    stage(1, bufs1)

    def pairstep(p, carry):
        c0 = 2 * p
        wait_stage(bufs1)
        fire(bufs1)

        @pl.when(p < NPAIR - 1)
        def _():
            stage(c0 + 2, bufs0)

        wait_g(bufs0)
        compute(c0, bufs0)

        @pl.when(p < NPAIR - 1)
        def _():
            wait_stage(bufs0)
            fire(bufs0)
            stage(c0 + 3, bufs1)

        wait_g(bufs1)
        compute(c0 + 1, bufs1)
        return carry

    lax.fori_loop(0, NPAIR, pairstep, 0)


def _sc_pred(inp, pos, neg, emb_w, emb_f_w, emb_r_w):
    mesh = plsc.VectorSubcoreMesh(core_axis_name="c", subcore_axis_name="s")
    buf_set = [
        pltpu.VMEM((R * SEQ,), jnp.int32),
        pltpu.VMEM((R * NPOS,), jnp.int32),
        pltpu.VMEM((R * NNEG,), jnp.int32),
        pltpu.VMEM((16,), jnp.int32),
        pltpu.VMEM((R * NA,), jnp.int32),
        pltpu.VMEM((R * NB,), jnp.int32),
        pltpu.VMEM((R, D2), jnp.float32),
        pltpu.VMEM((R * NA, 64), jnp.float32),
        pltpu.VMEM((R * NB, 64), jnp.float32),
        pltpu.VMEM((R * NPOS, D2), jnp.float32),
        pltpu.VMEM((R * NNEG, D2), jnp.float32),
        pltpu.VMEM((R * NCAND,), jnp.float32),
    ]
    fn = functools.partial(
        pl.kernel,
        out_type=jax.ShapeDtypeStruct((B * NCAND,), jnp.float32),
        scratch_types=buf_set + buf_set + [
            pltpu.SemaphoreType.DMA,
            pltpu.SemaphoreType.DMA,
        ],
        mesh=mesh,
        compiler_params=pltpu.CompilerParams(needs_layout_passes=False,
                                             use_tc_tiling_on_sc=False),
    )(_sc_body)
    return fn(inp, pos, neg, emb_w, emb_f_w, emb_r_w)


def _loss_body(pred_ref, out_ref):
    x = pred_ref[...]
    i = lax.broadcasted_iota(jnp.int32, x.shape, 0)
    j = lax.broadcasted_iota(jnp.int32, x.shape, 1)
    col = (i * 128 + j) % NCAND
    label = (col < NPOS).astype(jnp.float32)
    p = jnp.clip(jax.nn.sigmoid(x), 1e-7, 1.0 - 1e-7)
    ll = label * jnp.log(p) + (1.0 - label) * jnp.log(1.0 - p)
    out_ref[0, 0] = -jnp.sum(ll) * (1.0 / (B * NCAND))


def _loss(pred_flat):
    out = pl.pallas_call(
        _loss_body,
        out_shape=jax.ShapeDtypeStruct((1, 1), jnp.float32),
        out_specs=pl.BlockSpec(memory_space=pltpu.SMEM),
    )(pred_flat.reshape(B * NCAND // 128, 128))
    return out[0, 0]


def kernel(inp, pos, neg, emb_w, emb_f_w, emb_r_w):
    inp = inp.astype(jnp.int32).reshape(-1)
    pos = pos.astype(jnp.int32).reshape(-1)
    neg = neg.astype(jnp.int32).reshape(-1)
    pred = _sc_pred(inp, pos, neg, emb_w, emb_f_w, emb_r_w)
    return _loss(pred)


# speedup vs baseline: 6.1359x; 1.0015x over previous
import functools

import jax
import jax.numpy as jnp
from jax import lax
from jax.experimental import pallas as pl
from jax.experimental.pallas import tpu as pltpu
from jax.experimental.pallas import tpu_sc as plsc

B = 4096
VOCAB_ = 100000
SEQ = 50
D2 = 128
NPOS = 3
NNEG = 25
NCAND = NPOS + NNEG
NW = 32
ROWS_PER_W = B // NW
R = 8
NCHUNK = ROWS_PER_W // R
NPAIR = NCHUNK // 2
NA = 16
NB = 32


def _sc_body(inp_hbm, pos_hbm, neg_hbm,
             embw_hbm, embf_hbm, embr_hbm, out_hbm,
             inp0, pos0, neg0, idxf0, idxa0, idxb0,
             f0, a0, b0, erp0, ern0, pred0,
             inp1, pos1, neg1, idxf1, idxa1, idxb1,
             f1, a1, b1, erp1, ern1, pred1,
             sem_g, sem_s):
    wid = lax.axis_index("s") * 2 + lax.axis_index("c")
    base = wid * ROWS_PER_W
    lane = lax.iota(jnp.int32, 16)
    last = lane == 15

    bufs0 = (inp0, pos0, neg0, idxf0, idxa0, idxb0,
             f0, a0, b0, erp0, ern0, pred0)
    bufs1 = (inp1, pos1, neg1, idxf1, idxa1, idxb1,
             f1, a1, b1, erp1, ern1, pred1)

    def stage(c, bufs):
        inp_v, pos_v, neg_v = bufs[0], bufs[1], bufs[2]
        row0 = base + c * R
        pltpu.async_copy(inp_hbm.at[pl.ds(row0 * SEQ, R * SEQ)],
                         inp_v, sem_s)
        pltpu.async_copy(pos_hbm.at[pl.ds(row0 * NPOS, R * NPOS)],
                         pos_v, sem_s)
        pltpu.async_copy(neg_hbm.at[pl.ds(row0 * NNEG, R * NNEG)],
                         neg_v, sem_s)

    def wait_stage(bufs):
        inp_v, pos_v, neg_v = bufs[0], bufs[1], bufs[2]
        pltpu.make_async_copy(inp_hbm.at[pl.ds(0, R * SEQ)],
                              inp_v, sem_s).wait()
        pltpu.make_async_copy(pos_hbm.at[pl.ds(0, R * NPOS)],
                              pos_v, sem_s).wait()
        pltpu.make_async_copy(neg_hbm.at[pl.ds(0, R * NNEG)],
                              neg_v, sem_s).wait()

    def fire(bufs):
        (inp_v, pos_v, neg_v, idxf_v, idxa_v, idxb_v,
         f_v, a_v, b_v, erp_v, ern_v, _) = bufs
        idxf_v[pl.ds(0, 16)] = plsc.load_gather(inp_v, [(lane & 7) * SEQ])
        for r in range(R):
            idxa_v[pl.ds(r * NA, 16)] = plsc.load_gather(
                inp_v, [r * SEQ + 3 * lane + 1])
            for h in range(2):
                j = lane + h * 16
                k = j & 3
                idxb_v[pl.ds(r * NB + h * 16, 16)] = plsc.load_gather(
                    inp_v, [r * SEQ + 6 * (j >> 2) + 2 + k + (k >> 1)])
        pltpu.async_copy(embf_hbm.at[idxf_v.at[pl.ds(0, 8)]], f_v, sem_g)
        pltpu.async_copy(embw_hbm.at[idxa_v], a_v, sem_g)
        pltpu.async_copy(embw_hbm.at[idxb_v.at[pl.ds(0, 128)]],
                         b_v.at[pl.ds(0, 128)], sem_g)
        pltpu.async_copy(embw_hbm.at[idxb_v.at[pl.ds(128, 128)]],
                         b_v.at[pl.ds(128, 128)], sem_g)
        pltpu.async_copy(embr_hbm.at[pos_v], erp_v, sem_g)
        pltpu.async_copy(embr_hbm.at[neg_v.at[pl.ds(0, 104)]],
                         ern_v.at[pl.ds(0, 104)], sem_g)
        pltpu.async_copy(embr_hbm.at[neg_v.at[pl.ds(104, 96)]],
                         ern_v.at[pl.ds(104, 96)], sem_g)

    def wait_g(bufs):
        (_, pos_v, neg_v, idxf_v, idxa_v, idxb_v,
         f_v, a_v, b_v, erp_v, ern_v, _) = bufs
        pltpu.make_async_copy(embf_hbm.at[idxf_v.at[pl.ds(0, 8)]],
                              f_v, sem_g).wait()
        pltpu.make_async_copy(embw_hbm.at[idxa_v], a_v, sem_g).wait()
        pltpu.make_async_copy(embw_hbm.at[idxb_v.at[pl.ds(0, 128)]],
                              b_v.at[pl.ds(0, 128)], sem_g).wait()
        pltpu.make_async_copy(embw_hbm.at[idxb_v.at[pl.ds(128, 128)]],
                              b_v.at[pl.ds(128, 128)], sem_g).wait()
        pltpu.make_async_copy(embr_hbm.at[pos_v], erp_v, sem_g).wait()
        pltpu.make_async_copy(embr_hbm.at[neg_v.at[pl.ds(0, 104)]],
                              ern_v.at[pl.ds(0, 104)], sem_g).wait()
        pltpu.make_async_copy(embr_hbm.at[neg_v.at[pl.ds(104, 96)]],
                              ern_v.at[pl.ds(104, 96)], sem_g).wait()

    def compute(c, bufs):
        (_, _, _, _, _, _, f_v, a_v, b_v, erp_v, ern_v, pred_v) = bufs
        row0 = base + c * R

        def tree_sum(vs):
            while len(vs) > 1:
                nxt = [vs[i] + vs[i + 1] for i in range(0, len(vs) - 1, 2)]
                if len(vs) % 2:
                    nxt.append(vs[-1])
                vs = nxt
            return vs[0]

        def rowbody(r, carry):
            accA = [tree_sum([a_v[r * NA + j, pl.ds(q * 16, 16)]
                              for j in range(NA)]) for q in range(4)]
            accB = [tree_sum([b_v[r * NB + j, pl.ds(q * 16, 16)]
                              for j in range(NB)]) for q in range(4)]
            vv = []
            for q in range(4):
                vv.append((f_v[r, pl.ds(q * 16, 16)]
                           + accA[q] * 0.125) * (1.0 / 3.0))
            for q in range(4):
                vv.append((f_v[r, pl.ds(64 + q * 16, 16)]
                           + accB[q] * 0.0625) * (1.0 / 3.0))
            for g in range(0, NCAND, 4):
                accs = []
                for n in range(g, min(g + 4, NCAND)):
                    if n < NPOS:
                        er, i = erp_v, r * NPOS + n
                    else:
                        er, i = ern_v, r * NNEG + (n - NPOS)
                    accs.append(tree_sum(
                        [er[i, pl.ds(q * 16, 16)] * vv[q]
                         for q in range(8)]))
                tots = [plsc.cumsum(a) for a in accs]
                for k, tot in enumerate(tots):
                    plsc.store_scatter(
                        pred_v,
                        [jnp.full((16,), r * NCAND + g + k, jnp.int32)],
                        tot, mask=last)
            return carry

        lax.fori_loop(0, R, rowbody, 0)
        pltpu.sync_copy(pred_v, out_hbm.at[pl.ds(row0 * NCAND, R * NCAND)])

    stage(0, bufs0)
    wait_stage(bufs0)
    fire(bufs0)
    stage(1, bufs1)

    def pairstep(p, carry):
        c0 = 2 * p
        wait_stage(bufs1)
        fire(bufs1)

        @pl.when(p < NPAIR - 1)
        def _():
            stage(c0 + 2, bufs0)

        wait_g(bufs0)
        compute(c0, bufs0)

        @pl.when(p < NPAIR - 1)
        def _():
            wait_stage(bufs0)
            fire(bufs0)
            stage(c0 + 3, bufs1)

        wait_g(bufs1)
        compute(c0 + 1, bufs1)
        return carry

    lax.fori_loop(0, NPAIR, pairstep, 0)


def _sc_pred(inp, pos, neg, emb_w, emb_f_w, emb_r_w):
    mesh = plsc.VectorSubcoreMesh(core_axis_name="c", subcore_axis_name="s")
    buf_set = [
        pltpu.VMEM((R * SEQ,), jnp.int32),
        pltpu.VMEM((R * NPOS,), jnp.int32),
        pltpu.VMEM((R * NNEG,), jnp.int32),
        pltpu.VMEM((16,), jnp.int32),
        pltpu.VMEM((R * NA,), jnp.int32),
        pltpu.VMEM((R * NB,), jnp.int32),
        pltpu.VMEM((R, D2), jnp.float32),
        pltpu.VMEM((R * NA, 64), jnp.float32),
        pltpu.VMEM((R * NB, 64), jnp.float32),
        pltpu.VMEM((R * NPOS, D2), jnp.float32),
        pltpu.VMEM((R * NNEG, D2), jnp.float32),
        pltpu.VMEM((R * NCAND,), jnp.float32),
    ]
    fn = functools.partial(
        pl.kernel,
        out_type=jax.ShapeDtypeStruct((B * NCAND,), jnp.float32),
        scratch_types=buf_set + buf_set + [
            pltpu.SemaphoreType.DMA,
            pltpu.SemaphoreType.DMA,
        ],
        mesh=mesh,
        compiler_params=pltpu.CompilerParams(needs_layout_passes=False,
                                             use_tc_tiling_on_sc=False),
    )(_sc_body)
    return fn(inp, pos, neg, emb_w, emb_f_w, emb_r_w)


def _loss_body(pred_ref, out_ref):
    x = pred_ref[...]
    i = lax.broadcasted_iota(jnp.int32, x.shape, 0)
    j = lax.broadcasted_iota(jnp.int32, x.shape, 1)
    col = (i * 128 + j) % NCAND
    label = (col < NPOS).astype(jnp.float32)
    p = jnp.clip(jax.nn.sigmoid(x), 1e-7, 1.0 - 1e-7)
    ll = label * jnp.log(p) + (1.0 - label) * jnp.log(1.0 - p)
    out_ref[0, 0] = -jnp.sum(ll) * (1.0 / (B * NCAND))


def _loss(pred_flat):
    out = pl.pallas_call(
        _loss_body,
        out_shape=jax.ShapeDtypeStruct((1, 1), jnp.float32),
        out_specs=pl.BlockSpec(memory_space=pltpu.SMEM),
    )(pred_flat.reshape(B * NCAND // 128, 128))
    return out[0, 0]


def kernel(inp, pos, neg, emb_w, emb_f_w, emb_r_w):
    inp = inp.astype(jnp.int32).reshape(-1)
    pos = pos.astype(jnp.int32).reshape(-1)
    neg = neg.astype(jnp.int32).reshape(-1)
    emb_w = emb_w.reshape(-1).reshape(VOCAB_, 64)
    pred = _sc_pred(inp, pos, neg, emb_w, emb_f_w, emb_r_w)
    return _loss(pred)
